# baseline (speedup 1.0000x reference)
import functools

import jax
import jax.numpy as jnp
from jax import lax
from jax.experimental import pallas as pl
from jax.experimental.pallas import tpu as pltpu
from jax.experimental.pallas import tpu_sc as plsc

S = 2048
D = 768
FF = 3072
E = 8
CAP = 320
SLOT_PER_E = 328
NSLOT = E * SLOT_PER_E
FF_BLK = 1536
N_FF = FF // FF_BLK

NC = 2
NS = 16
NW = NC * NS
TOK_PER_W = S // NW


def _router_body(hs_ref, wcls_ref, logits_ref, slot_ref, p_ref, fei_ref,
                 routed_ref):
    logits = jnp.dot(hs_ref[...], wcls_ref[...],
                     preferred_element_type=jnp.float32)
    logits_ref[...] = logits
    lmax = jnp.max(logits, axis=1, keepdims=True)
    p_ref[...] = 1.0 / jnp.sum(jnp.exp(logits - lmax), axis=1, keepdims=True)
    col = lax.broadcasted_iota(jnp.int32, (S, E), 1)
    eidx = jnp.min(jnp.where(logits == lmax, col, E), axis=1,
                   keepdims=True)
    onehot = (col == eidx).astype(jnp.float32)
    cum = onehot
    k = 1
    while k < S:
        cum = cum + jnp.concatenate(
            [jnp.zeros((k, E), jnp.float32), cum[:S - k, :]], axis=0)
        k *= 2
    prio = jnp.sum(cum * onehot, axis=1, keepdims=True)
    routed = prio <= float(CAP)
    rank = (prio - 1.0).astype(jnp.int32)
    slot_ref[...] = eidx * SLOT_PER_E + jnp.where(routed, rank, CAP)
    fei_ref[...] = jnp.where(routed, eidx, 0)
    routed_ref[...] = routed.astype(jnp.int32)


def _router_call(hs, wcls):
    return pl.pallas_call(
        _router_body,
        out_shape=(
            jax.ShapeDtypeStruct((S, E), jnp.float32),
            jax.ShapeDtypeStruct((S, 1), jnp.int32),
            jax.ShapeDtypeStruct((S, 1), jnp.float32),
            jax.ShapeDtypeStruct((S, 1), jnp.int32),
            jax.ShapeDtypeStruct((S, 1), jnp.int32),
        ),
    )(hs, wcls)


@functools.cache
def _make_dispatch():
    mesh = plsc.VectorSubcoreMesh(core_axis_name="c", subcore_axis_name="s")

    @functools.partial(
        pl.kernel, mesh=mesh,
        out_type=jax.ShapeDtypeStruct((NSLOT, D), jnp.float32),
        scratch_types=[
            pltpu.VMEM((TOK_PER_W,), jnp.int32),
            pltpu.VMEM((TOK_PER_W, D), jnp.float32),
            pltpu.SemaphoreType.DMA,
        ],
    )
    def _dispatch(hs_hbm, slot_hbm, xbuf_hbm, idx_v, rows_v, sem):
        wid = lax.axis_index("s") * NC + lax.axis_index("c")
        base = wid * TOK_PER_W
        pltpu.sync_copy(slot_hbm.at[pl.ds(base, TOK_PER_W)], idx_v)
        pltpu.sync_copy(hs_hbm.at[pl.ds(base, TOK_PER_W)], rows_v)
        pltpu.async_copy(rows_v, xbuf_hbm.at[idx_v], sem).wait()

    return _dispatch


def _ffn_body(x_ref, wi_ref, wo_ref, y_ref):
    f = pl.program_id(1)
    h = jnp.maximum(
        jnp.dot(x_ref[...], wi_ref[0], preferred_element_type=jnp.float32),
        0.0)
    part = jnp.dot(h, wo_ref[0], preferred_element_type=jnp.float32)

    @pl.when(f == 0)
    def _():
        y_ref[...] = part

    @pl.when(f > 0)
    def _():
        y_ref[...] += part


def _ffn_call(xbuf, wi, wo):
    return pl.pallas_call(
        _ffn_body,
        grid=(E, N_FF),
        in_specs=[
            pl.BlockSpec((SLOT_PER_E, D), lambda e, f: (e, 0)),
            pl.BlockSpec((1, D, FF_BLK), lambda e, f: (e, 0, f)),
            pl.BlockSpec((1, FF_BLK, D), lambda e, f: (e, f, 0)),
        ],
        out_specs=pl.BlockSpec((SLOT_PER_E, D), lambda e, f: (e, 0)),
        out_shape=jax.ShapeDtypeStruct((NSLOT, D), jnp.float32),
    )(xbuf, wi, wo)


@functools.cache
def _make_collect():
    mesh = plsc.VectorSubcoreMesh(core_axis_name="c", subcore_axis_name="s")

    @functools.partial(
        pl.kernel, mesh=mesh,
        out_type=jax.ShapeDtypeStruct((S, D), jnp.float32),
        scratch_types=[
            pltpu.VMEM((TOK_PER_W,), jnp.int32),
            pltpu.VMEM((TOK_PER_W, D), jnp.float32),
            pltpu.SemaphoreType.DMA,
        ],
    )
    def _collect(y_hbm, slot_hbm, yg_hbm, idx_v, rows_v, sem):
        wid = lax.axis_index("s") * NC + lax.axis_index("c")
        base = wid * TOK_PER_W
        pltpu.sync_copy(slot_hbm.at[pl.ds(base, TOK_PER_W)], idx_v)
        pltpu.async_copy(y_hbm.at[idx_v], rows_v, sem).wait()
        pltpu.sync_copy(rows_v, yg_hbm.at[pl.ds(base, TOK_PER_W)])

    return _collect


def _combine_body(yg_ref, hs_ref, p_ref, routed_ref, out_ref):
    sel = routed_ref[...] > 0
    out_ref[...] = p_ref[...] * jnp.where(sel, yg_ref[...], hs_ref[...])


def _combine_call(yg, hs, p, routed):
    blk = 256
    return pl.pallas_call(
        _combine_body,
        grid=(S // blk,),
        in_specs=[
            pl.BlockSpec((blk, D), lambda i: (i, 0)),
            pl.BlockSpec((blk, D), lambda i: (i, 0)),
            pl.BlockSpec((blk, 1), lambda i: (i, 0)),
            pl.BlockSpec((blk, 1), lambda i: (i, 0)),
        ],
        out_specs=pl.BlockSpec((blk, D), lambda i: (i, 0)),
        out_shape=jax.ShapeDtypeStruct((S, D), jnp.float32),
    )(yg, hs, p, routed)


def kernel(hidden_states, W_cls, wi, wo):
    hs = hidden_states.reshape(S, D)
    logits, slot, p, fei, routed = _router_call(hs, W_cls)
    slot_flat = slot.reshape(S)
    xbuf = _make_dispatch()(hs, slot_flat)
    y = _ffn_call(xbuf, wi, wo)
    yg = _make_collect()(y, slot_flat)
    out = _combine_call(yg, hs, p, routed)
    return (out.reshape(1, S, D),
            logits.reshape(1, S, E),
            fei.reshape(1, S))


# speedup vs baseline: 1.1157x; 1.1157x over previous
import functools

import jax
import jax.numpy as jnp
from jax import lax
from jax.experimental import pallas as pl
from jax.experimental.pallas import tpu as pltpu
from jax.experimental.pallas import tpu_sc as plsc

S = 2048
D = 768
FF = 3072
E = 8
CAP = 320
SLOT_PER_E = 328
NSLOT = E * SLOT_PER_E
NROW = NSLOT + S

NC = 2
NS = 16
NW = NC * NS
TOK_PER_W = S // NW


def _router_body(hs_ref, wcls_ref, logits_ref, slot_ref, fei_ref, xs_ref):
    logits = jnp.dot(hs_ref[...], wcls_ref[...],
                     preferred_element_type=jnp.float32)
    logits_ref[...] = logits
    lmax = jnp.max(logits, axis=1, keepdims=True)
    p = 1.0 / jnp.sum(jnp.exp(logits - lmax), axis=1, keepdims=True)
    xs_ref[...] = p * hs_ref[...]
    col = lax.broadcasted_iota(jnp.int32, (S, E), 1)
    eidx = jnp.min(jnp.where(logits == lmax, col, E), axis=1,
                   keepdims=True)
    onehot = (col == eidx).astype(jnp.float32)
    cum = onehot
    k = 1
    while k < S:
        cum = cum + jnp.concatenate(
            [jnp.zeros((k, E), jnp.float32), cum[:S - k, :]], axis=0)
        k *= 2
    prio = jnp.sum(cum * onehot, axis=1, keepdims=True)
    routed = prio <= float(CAP)
    rank = (prio - 1.0).astype(jnp.int32)
    row = lax.broadcasted_iota(jnp.int32, (S, 1), 0)
    slot_ref[...] = jnp.where(routed, eidx * SLOT_PER_E + rank, NSLOT + row)
    fei_ref[...] = jnp.where(routed, eidx, 0)


def _router_call(hs, wcls):
    return pl.pallas_call(
        _router_body,
        out_shape=(
            jax.ShapeDtypeStruct((S, E), jnp.float32),
            jax.ShapeDtypeStruct((S, 1), jnp.int32),
            jax.ShapeDtypeStruct((S, 1), jnp.int32),
            jax.ShapeDtypeStruct((S, D), jnp.float32),
        ),
    )(hs, wcls)


@functools.cache
def _make_dispatch():
    mesh = plsc.VectorSubcoreMesh(core_axis_name="c", subcore_axis_name="s")

    @functools.partial(
        pl.kernel, mesh=mesh,
        out_type=jax.ShapeDtypeStruct((NROW, D), jnp.float32),
        scratch_types=[
            pltpu.VMEM((TOK_PER_W,), jnp.int32),
            pltpu.VMEM((TOK_PER_W, D), jnp.float32),
            pltpu.SemaphoreType.DMA,
        ],
    )
    def _dispatch(xs_hbm, slot_hbm, xbuf_hbm, idx_v, rows_v, sem):
        wid = lax.axis_index("s") * NC + lax.axis_index("c")
        base = wid * TOK_PER_W
        pltpu.sync_copy(slot_hbm.at[pl.ds(base, TOK_PER_W)], idx_v)
        pltpu.sync_copy(xs_hbm.at[pl.ds(base, TOK_PER_W)], rows_v)
        pltpu.async_copy(rows_v, xbuf_hbm.at[idx_v], sem).wait()

    return _dispatch


def _ffn_body(x_ref, wi_ref, wo_ref, y_ref):
    h = jnp.maximum(
        jnp.dot(x_ref[...], wi_ref[0], preferred_element_type=jnp.float32),
        0.0)
    y_ref[...] = jnp.dot(h, wo_ref[0], preferred_element_type=jnp.float32)


def _ffn_call(xbuf, wi, wo):
    return pl.pallas_call(
        _ffn_body,
        grid=(E,),
        in_specs=[
            pl.BlockSpec((SLOT_PER_E, D), lambda e: (e, 0)),
            pl.BlockSpec((1, D, FF), lambda e: (e, 0, 0)),
            pl.BlockSpec((1, FF, D), lambda e: (e, 0, 0)),
        ],
        out_specs=pl.BlockSpec((SLOT_PER_E, D), lambda e: (e, 0)),
        out_shape=jax.ShapeDtypeStruct((NROW, D), jnp.float32),
        input_output_aliases={0: 0},
    )(xbuf, wi, wo)


@functools.cache
def _make_collect():
    mesh = plsc.VectorSubcoreMesh(core_axis_name="c", subcore_axis_name="s")

    @functools.partial(
        pl.kernel, mesh=mesh,
        out_type=jax.ShapeDtypeStruct((S, D), jnp.float32),
        scratch_types=[
            pltpu.VMEM((TOK_PER_W,), jnp.int32),
            pltpu.VMEM((TOK_PER_W, D), jnp.float32),
            pltpu.SemaphoreType.DMA,
        ],
    )
    def _collect(y_hbm, slot_hbm, out_hbm, idx_v, rows_v, sem):
        wid = lax.axis_index("s") * NC + lax.axis_index("c")
        base = wid * TOK_PER_W
        pltpu.sync_copy(slot_hbm.at[pl.ds(base, TOK_PER_W)], idx_v)
        pltpu.async_copy(y_hbm.at[idx_v], rows_v, sem).wait()
        pltpu.sync_copy(rows_v, out_hbm.at[pl.ds(base, TOK_PER_W)])

    return _collect


def kernel(hidden_states, W_cls, wi, wo):
    hs = hidden_states.reshape(S, D)
    logits, slot, fei, xs = _router_call(hs, W_cls)
    slot_flat = slot.reshape(S)
    xbuf = _make_dispatch()(xs, slot_flat)
    y = _ffn_call(xbuf, wi, wo)
    out = _make_collect()(y, slot_flat)
    return (out.reshape(1, S, D),
            logits.reshape(1, S, E),
            fei.reshape(1, S))
